# baseline (speedup 1.0000x reference)
import jax
import jax.numpy as jnp
from jax import lax
from jax.experimental import pallas as pl
from jax.experimental.pallas import tpu as pltpu
from jax.experimental.pallas import tpu_sc as plsc

D = 1024
BATCH = 4
SEQ = 2048
NSC = 2
NTILE = 16
RPT = SEQ // (NSC * NTILE)
P = 8
NSUB = RPT // P
NCH = BATCH * NSUB


NSLOT = 6
KPRE = NSLOT - 3


def _sc_body(x_hbm, pos_hbm, out_hbm, pbuf, xbuf, spos, sin, sout):
    sc = lax.axis_index("c")
    t = lax.axis_index("s")
    row0 = sc * (NTILE * RPT) + t * RPT

    def loc(c):
        return c // NSUB, row0 + lax.rem(c, NSUB) * P, lax.rem(c, NSLOT)

    def start_in(c):
        b, r, slot = loc(c)
        pltpu.async_copy(x_hbm.at[b, pl.ds(r, P), :], xbuf.at[slot], sin.at[slot])

    def start_out(c):
        b, r, slot = loc(c)
        pltpu.async_copy(xbuf.at[slot], out_hbm.at[b, pl.ds(r, P), :], sout.at[slot])

    def wait_in(c):
        slot = lax.rem(c, NSLOT)
        pltpu.make_async_copy(
            x_hbm.at[0, pl.ds(0, P), :], xbuf.at[slot], sin.at[slot]
        ).wait()

    def wait_out(c):
        slot = lax.rem(c, NSLOT)
        pltpu.make_async_copy(
            xbuf.at[slot], out_hbm.at[0, pl.ds(0, P), :], sout.at[slot]
        ).wait()

    def add(c):
        slot = lax.rem(c, NSLOT)
        sub = lax.rem(c, NSUB)

        @plsc.parallel_loop(0, P)
        def _(i):
            prow = sub * P + i
            for j in range(0, D, 16):
                sl = pl.ds(j, 16)
                plsc.addupdate(xbuf.at[slot, i, sl], pbuf[prow, sl])

    cpos = pltpu.async_copy(pos_hbm.at[pl.ds(row0, RPT), :], pbuf, spos)
    for c0 in range(KPRE):
        start_in(c0)
    cpos.wait()

    def body(c, carry):
        @pl.when(c + KPRE < NCH)
        def _():
            @pl.when(c + KPRE >= NSLOT)
            def _():
                wait_out(c + KPRE - NSLOT)

            start_in(c + KPRE)

        wait_in(c)
        add(c)
        start_out(c)
        return carry

    lax.fori_loop(0, NCH, body, 0)
    for c0 in range(NCH - NSLOT, NCH):
        wait_out(c0)


@jax.jit
def _sc_add(x, pos_table):
    mesh = plsc.VectorSubcoreMesh(core_axis_name="c", subcore_axis_name="s")
    return pl.kernel(
        _sc_body,
        out_type=jax.ShapeDtypeStruct((BATCH, SEQ, D), jnp.float32),
        mesh=mesh,
        scratch_types=[
            pltpu.VMEM((RPT, D), jnp.float32),
            pltpu.VMEM((NSLOT, P, D), jnp.float32),
            pltpu.SemaphoreType.DMA,
            pltpu.SemaphoreType.DMA((NSLOT,)),
            pltpu.SemaphoreType.DMA((NSLOT,)),
        ],
        compiler_params=pltpu.CompilerParams(use_tc_tiling_on_sc=True),
    )(x, pos_table)


def kernel(x, pos_table):
    return _sc_add(x, pos_table)


# speedup vs baseline: 1.0097x; 1.0097x over previous
import jax
import jax.numpy as jnp
from jax import lax
from jax.experimental import pallas as pl
from jax.experimental.pallas import tpu as pltpu
from jax.experimental.pallas import tpu_sc as plsc

D = 1024
BATCH = 4
SEQ = 2048
NSC = 2
NTILE = 16
RPT = SEQ // (NSC * NTILE)
P = 8
NSUB = RPT // P
NCH = BATCH * NSUB


NSLOT = 6
KPRE = NSLOT - 2


def _sc_body(x_hbm, pos_hbm, out_hbm, pbuf, xbuf, spos, sin, sout):
    sc = lax.axis_index("c")
    t = lax.axis_index("s")
    row0 = sc * (NTILE * RPT) + t * RPT

    def loc(c):
        return c // NSUB, row0 + lax.rem(c, NSUB) * P, lax.rem(c, NSLOT)

    def start_in(c):
        b, r, slot = loc(c)
        pltpu.async_copy(x_hbm.at[b, pl.ds(r, P), :], xbuf.at[slot], sin.at[slot])

    def start_out(c):
        b, r, slot = loc(c)
        pltpu.async_copy(xbuf.at[slot], out_hbm.at[b, pl.ds(r, P), :], sout.at[slot])

    def wait_in(c):
        slot = lax.rem(c, NSLOT)
        pltpu.make_async_copy(
            x_hbm.at[0, pl.ds(0, P), :], xbuf.at[slot], sin.at[slot]
        ).wait()

    def wait_out(c):
        slot = lax.rem(c, NSLOT)
        pltpu.make_async_copy(
            xbuf.at[slot], out_hbm.at[0, pl.ds(0, P), :], sout.at[slot]
        ).wait()

    def add(c):
        slot = lax.rem(c, NSLOT)
        sub = lax.rem(c, NSUB)

        @plsc.parallel_loop(0, P)
        def _(i):
            prow = sub * P + i
            for j in range(0, D, 16):
                sl = pl.ds(j, 16)
                plsc.addupdate(xbuf.at[slot, i, sl], pbuf[prow, sl])

    cpos = pltpu.async_copy(pos_hbm.at[pl.ds(row0, RPT), :], pbuf, spos)
    for c0 in range(KPRE):
        start_in(c0)
    cpos.wait()

    def body(c, carry):
        @pl.when(c + KPRE < NCH)
        def _():
            @pl.when(c + KPRE >= NSLOT)
            def _():
                wait_out(c + KPRE - NSLOT)

            start_in(c + KPRE)

        wait_in(c)
        add(c)
        start_out(c)
        return carry

    lax.fori_loop(0, NCH, body, 0)
    for c0 in range(NCH - NSLOT, NCH):
        wait_out(c0)


@jax.jit
def _sc_add(x, pos_table):
    mesh = plsc.VectorSubcoreMesh(core_axis_name="c", subcore_axis_name="s")
    return pl.kernel(
        _sc_body,
        out_type=jax.ShapeDtypeStruct((BATCH, SEQ, D), jnp.float32),
        mesh=mesh,
        scratch_types=[
            pltpu.VMEM((RPT, D), jnp.float32),
            pltpu.VMEM((NSLOT, P, D), jnp.float32),
            pltpu.SemaphoreType.DMA,
            pltpu.SemaphoreType.DMA((NSLOT,)),
            pltpu.SemaphoreType.DMA((NSLOT,)),
        ],
        compiler_params=pltpu.CompilerParams(use_tc_tiling_on_sc=True),
    )(x, pos_table)


def kernel(x, pos_table):
    return _sc_add(x, pos_table)
